# baseline (speedup 1.0000x reference)
import functools

import jax
import jax.numpy as jnp
from jax import lax
from jax.experimental import pallas as pl
from jax.experimental.pallas import tpu as pltpu
from jax.experimental.pallas import tpu_sc as plsc

HASH_BUCKETS = 250000
EMBED_DIM = 64
LANES = 16
NW = 32
BCK_BLK = 128
N_FULL_BLK = HASH_BUCKETS // BCK_BLK
TAIL = HASH_BUCKETS - N_FULL_BLK * BCK_BLK


def _tr_body(tt_hbm, tail_hbm, r_hbm, in0, in1, out0, out1,
             rs0, rs1, ws0, ws1):
    cid = lax.axis_index("c")
    sid = lax.axis_index("s")
    wid = sid * 2 + cid
    iota = lax.iota(jnp.int32, LANES)
    inb = (in0, in1)
    outb = (out0, out1)
    rsem = (rs0, rs1)
    wsem = (ws0, ws1)
    NBLK = 62

    def b_of(k):
        return jnp.minimum(wid + k * NW, N_FULL_BLK - 1) * BCK_BLK

    def fire_read(k, p):
        pltpu.async_copy(tt_hbm.at[:, pl.ds(b_of(k), BCK_BLK)], inb[p],
                         rsem[p])

    def wait_read(p):
        pltpu.make_async_copy(tt_hbm.at[:, pl.ds(0, BCK_BLK)], inb[p],
                              rsem[p]).wait()

    def transpose(p):
        @plsc.parallel_loop(0, EMBED_DIM, unroll=8)
        def drow(d):
            col = jnp.full((LANES,), 0, jnp.int32) + d
            for c0 in range(0, BCK_BLK, LANES):
                v = inb[p][d, pl.ds(c0, LANES)]
                plsc.store_scatter(outb[p], [iota + c0, col], v)

    def fire_write(k, p):
        pltpu.async_copy(outb[p], r_hbm.at[pl.ds(b_of(k), BCK_BLK)], wsem[p])

    def wait_write(p):
        pltpu.make_async_copy(r_hbm.at[pl.ds(0, BCK_BLK)], outb[p],
                              wsem[p]).wait()

    fire_read(0, 0)
    fire_read(1, 1)
    for p in range(2):
        wait_read(p)
        transpose(p)
        fire_read(2 + p, p)
        fire_write(p, p)

    def pair(q, carry):
        for p in range(2):
            k = 2 * q + p
            wait_read(p)
            wait_write(p)
            transpose(p)
            fire_read(k + 2, p)
            fire_write(k, p)
        return carry

    lax.fori_loop(1, NBLK // 2 - 1, pair, 0)

    for p in range(2):
        k = NBLK - 2 + p
        wait_read(p)
        wait_write(p)
        transpose(p)
        fire_write(k, p)
    for p in range(2):
        wait_write(p)

    @pl.when(wid == NW - 1)
    def _tail():
        b0 = N_FULL_BLK * BCK_BLK
        pltpu.sync_copy(tail_hbm, out0.at[pl.ds(0, TAIL)])
        pltpu.sync_copy(out0.at[pl.ds(0, TAIL)], r_hbm.at[pl.ds(b0, TAIL)])


def _gather_unit(xt_hbm, r_hbm, o_hbm, idxb, grows, obuf, gsem, wsem,
                 iota, rowv, j0, i0, nfeat):
    pltpu.sync_copy(xt_hbm.at[pl.ds(j0, nfeat), pl.ds(i0, 128)],
                    idxb.at[pl.ds(0, nfeat)])

    def mod_one(t, carry):
        r = t // 8
        c0 = (t % 8) * LANES
        idxb[r, pl.ds(c0, LANES)] = lax.rem(idxb[r, pl.ds(c0, LANES)],
                                            HASH_BUCKETS)
        return carry

    lax.fori_loop(0, nfeat * 8, mod_one, 0)

    def fire_f(f):
        pltpu.async_copy(r_hbm.at[idxb.at[f]], grows[f % 4], gsem[f % 4])

    for f in range(min(3, nfeat)):
        fire_f(f)
    for f in range(nfeat):
        if f + 3 < nfeat:
            fire_f(f + 3)
        pltpu.make_async_copy(r_hbm.at[pl.ds(0, 128)], grows[f % 4],
                              gsem[f % 4]).wait()

        ob = obuf[f % 2]
        g = grows[f % 4]

        if f >= 2:
            pltpu.make_async_copy(r_hbm.at[pl.ds(0, 64)], ob,
                                  wsem[f % 2]).wait()

        @plsc.parallel_loop(0, 128, unroll=16)
        def tr(i):
            col = jnp.full((LANES,), 0, jnp.int32) + i
            for t in range(4):
                v = g[i, pl.ds(t * LANES, LANES)]
                plsc.store_scatter(ob, [rowv[t], col], v)

        pltpu.async_copy(ob, o_hbm.at[j0 + f, :, pl.ds(i0, 128)],
                         wsem[f % 2])
    for f in range(max(nfeat - 2, 0), nfeat):
        pltpu.make_async_copy(r_hbm.at[pl.ds(0, 64)], obuf[f % 2],
                              wsem[f % 2]).wait()


def _gather_body(xt_hbm, r_hbm, o_hbm, idxb, g0, g1, g2, g3, ob0, ob1,
                 gsem0, gsem1, gsem2, gsem3, wsem0, wsem1):
    cid = lax.axis_index("c")
    sid = lax.axis_index("s")
    wid = sid * 2 + cid
    ibase = wid * 512
    iota = lax.iota(jnp.int32, LANES)
    rowv = tuple(iota + t * LANES for t in range(4))
    grows = (g0, g1, g2, g3)
    obuf = (ob0, ob1)
    gsem = (gsem0, gsem1, gsem2, gsem3)
    wsem = (wsem0, wsem1)

    def unit(u, carry):
        jt = u // 4
        iblk = u - jt * 4
        _gather_unit(xt_hbm, r_hbm, o_hbm, idxb, grows, obuf, gsem, wsem,
                     iota, rowv, jt * 8, ibase + iblk * 128, 8)
        return carry

    lax.fori_loop(0, 48, unit, 0)

    for iblk in range(4):
        _gather_unit(xt_hbm, r_hbm, o_hbm, idxb, grows, obuf, gsem, wsem,
                     iota, rowv, 96, ibase + iblk * 128, 4)


def kernel(x, table):
    xt = x.T
    tt = table.T
    tail = lax.dynamic_slice(table, (N_FULL_BLK * BCK_BLK, 0), (TAIL, EMBED_DIM))
    tail_p = jnp.pad(tail, ((0, 0), (0, 128 - EMBED_DIM)))
    mesh = plsc.VectorSubcoreMesh(core_axis_name="c", subcore_axis_name="s")
    cp = pltpu.CompilerParams(use_tc_tiling_on_sc=True, needs_layout_passes=False)

    tr = functools.partial(
        pl.kernel,
        mesh=mesh,
        compiler_params=cp,
        out_type=jax.ShapeDtypeStruct((HASH_BUCKETS, 128), jnp.float32),
        scratch_types=[
            pltpu.VMEM((EMBED_DIM, BCK_BLK), jnp.float32),
            pltpu.VMEM((EMBED_DIM, BCK_BLK), jnp.float32),
            pltpu.VMEM((BCK_BLK, 128), jnp.float32),
            pltpu.VMEM((BCK_BLK, 128), jnp.float32),
            pltpu.SemaphoreType.DMA,
            pltpu.SemaphoreType.DMA,
            pltpu.SemaphoreType.DMA,
            pltpu.SemaphoreType.DMA,
        ],
    )(_tr_body)
    r = tr(tt, tail_p)

    ga = functools.partial(
        pl.kernel,
        mesh=mesh,
        compiler_params=cp,
        out_type=jax.ShapeDtypeStruct((100, EMBED_DIM, 16384), jnp.float32),
        scratch_types=[
            pltpu.VMEM((8, 128), jnp.int32),
            pltpu.VMEM((128, 128), jnp.float32),
            pltpu.VMEM((128, 128), jnp.float32),
            pltpu.VMEM((128, 128), jnp.float32),
            pltpu.VMEM((128, 128), jnp.float32),
            pltpu.VMEM((EMBED_DIM, 128), jnp.float32),
            pltpu.VMEM((EMBED_DIM, 128), jnp.float32),
            pltpu.SemaphoreType.DMA,
            pltpu.SemaphoreType.DMA,
            pltpu.SemaphoreType.DMA,
            pltpu.SemaphoreType.DMA,
            pltpu.SemaphoreType.DMA,
            pltpu.SemaphoreType.DMA,
        ],
    )(_gather_body)
    o = ga(xt, r)
    return jnp.transpose(o, (2, 0, 1))


# speedup vs baseline: 1.3940x; 1.3940x over previous
import functools

import jax
import jax.numpy as jnp
from jax import lax
from jax.experimental import pallas as pl
from jax.experimental.pallas import tpu as pltpu
from jax.experimental.pallas import tpu_sc as plsc

HASH_BUCKETS = 250000
EMBED_DIM = 64
CHUNK = 128
K = 4
GROUP = K * CHUNK
LANES = 16
NW = 32


def _emb_body(n_groups, x_hbm, table_hbm, out_hbm,
              idx_a, idx_b, rows_a, rows_b,
              gsem_a, gsem_b, wsem_a, wsem_b):
    wid = lax.axis_index("s") * 2 + lax.axis_index("c")
    base_row = wid * (n_groups * GROUP)

    def load_idx(idx_s, g):
        row0 = base_row + g * GROUP
        pltpu.sync_copy(x_hbm.at[pl.ds(row0, GROUP)], idx_s)

        def mod_one(j, carry):
            sl = pl.ds(j * LANES, LANES)
            idx_s[sl] = lax.rem(idx_s[sl], HASH_BUCKETS)
            return carry

        lax.fori_loop(0, GROUP // LANES, mod_one, 0)

    def fire_gathers(idx_s, rows_s, gsem):
        for b in range(K):
            pltpu.async_copy(table_hbm.at[idx_s.at[pl.ds(b * CHUNK, CHUNK)]],
                             rows_s.at[pl.ds(b * CHUNK, CHUNK)], gsem)

    def drain_gathers(rows_s, gsem):
        pltpu.make_async_copy(out_hbm.at[pl.ds(base_row, GROUP)], rows_s,
                              gsem).wait()

    def fire_write(rows_s, wsem, g):
        row0 = base_row + g * GROUP
        pltpu.async_copy(rows_s, out_hbm.at[pl.ds(row0, GROUP)], wsem)

    def wait_write(rows_s, wsem):
        pltpu.make_async_copy(out_hbm.at[pl.ds(base_row, GROUP)], rows_s,
                              wsem).wait()

    load_idx(idx_a, 0)
    fire_gathers(idx_a, rows_a, gsem_a)
    load_idx(idx_b, 1)
    fire_gathers(idx_b, rows_b, gsem_b)

    def pair(i, carry):
        drain_gathers(rows_a, gsem_a)
        fire_write(rows_a, wsem_a, 2 * i)
        load_idx(idx_a, 2 * i + 2)
        wait_write(rows_a, wsem_a)
        fire_gathers(idx_a, rows_a, gsem_a)
        drain_gathers(rows_b, gsem_b)
        fire_write(rows_b, wsem_b, 2 * i + 1)
        load_idx(idx_b, 2 * i + 3)
        wait_write(rows_b, wsem_b)
        fire_gathers(idx_b, rows_b, gsem_b)
        return carry

    n_pairs = n_groups // 2
    lax.fori_loop(0, n_pairs - 1, pair, 0)

    drain_gathers(rows_a, gsem_a)
    fire_write(rows_a, wsem_a, n_groups - 2)
    drain_gathers(rows_b, gsem_b)
    fire_write(rows_b, wsem_b, n_groups - 1)
    wait_write(rows_a, wsem_a)
    wait_write(rows_b, wsem_b)


def kernel(x, table):
    rows, cols = x.shape
    b = rows * cols
    xf = x.reshape(b).astype(jnp.int32)
    assert b % (NW * GROUP) == 0
    n_groups = b // (NW * GROUP)
    assert n_groups % 2 == 0

    mesh = plsc.VectorSubcoreMesh(core_axis_name="c", subcore_axis_name="s")
    run = functools.partial(
        pl.kernel,
        mesh=mesh,
        compiler_params=pltpu.CompilerParams(use_tc_tiling_on_sc=False),
        out_type=jax.ShapeDtypeStruct((b, EMBED_DIM), jnp.float32),
        scratch_types=(
            [pltpu.VMEM((GROUP,), jnp.int32) for _ in range(2)]
            + [pltpu.VMEM((GROUP, EMBED_DIM), jnp.float32) for _ in range(2)]
            + [pltpu.SemaphoreType.DMA for _ in range(4)]
        ),
    )(functools.partial(_emb_body, n_groups))
    out = run(xf, table)
    return out.reshape(rows, cols, EMBED_DIM)
